# baseline (speedup 1.0000x reference)
import math

import jax
import jax.numpy as jnp
import numpy as np
from jax.experimental import pallas as pl

_L = 10
_NCOLS = _L * _L
_KDIM = 32
_BLK = 16384


def _build_table():
    from numpy.polynomial import chebyshev as _cheb
    Q = {(0, 0): np.zeros(_L)}
    Q[(0, 0)][0] = 1.0

    def shift(q):
        r = np.zeros(_L)
        r[1:] = q[:-1]
        return r

    for m in range(1, _L):
        Q[(m, m)] = -(2.0 * m - 1.0) * Q[(m - 1, m - 1)]
    for m in range(0, _L - 1):
        Q[(m + 1, m)] = (2.0 * m + 1.0) * shift(Q[(m, m)])
    for m in range(0, _L):
        for l in range(m + 2, _L):
            Q[(l, m)] = ((2.0 * l - 1.0) * shift(Q[(l - 1, m)])
                         - (l + m - 1.0) * Q[(l - 2, m)]) / (l - m)

    tab = np.zeros((_KDIM, 256))
    j = 0
    for l in range(_L):
        for m in range(-l, l + 1):
            am = abs(m)
            K = math.sqrt((2 * l + 1) / (4.0 * math.pi)
                          * math.factorial(l - am) / math.factorial(l + am))
            fac = K * (math.sqrt(2.0) if m != 0 else 1.0)
            cheb = _cheb.poly2cheb(fac * Q[(l, am)])
            tab[0:len(cheb), j] = cheb
            tab[(10 + am) if m >= 0 else (20 + am), 128 + j] = 1.0
            j += 1
    import ml_dtypes
    tab32 = tab.astype(np.float32)
    t_hi = tab32.astype(ml_dtypes.bfloat16)
    t_lo = (tab32 - t_hi.astype(np.float32)).astype(ml_dtypes.bfloat16)
    return np.concatenate([t_hi, t_lo, t_hi], axis=0)


_TAB96 = _build_table()


def _sph_body(ll_ref, t_ref, o_ref):
    lon = ll_ref[0, 0]
    lat = ll_ref[1, 0]
    deg = np.float32(math.pi / 180.0)
    phi = (lon + 180.0) * deg
    theta = (lat + 90.0) * deg
    x = jnp.cos(theta)
    sn = jnp.sin(theta)
    wr = sn * jnp.cos(phi)
    wi = sn * jnp.sin(phi)

    one = jnp.ones_like(x)
    zero = jnp.zeros_like(x)
    two_x = x + x
    ts = [one, x]
    for _ in range(2, _L):
        ts.append(two_x * ts[-1] - ts[-2])
    re = [one]
    im = [zero]
    for m in range(1, _L):
        re.append(re[m - 1] * wr - im[m - 1] * wi)
        im.append(re[m - 1] * wi + im[m - 1] * wr)

    F = jnp.stack(ts + re + im + [zero, zero], axis=0)
    f_hi = F.astype(jnp.bfloat16)
    f_lo = (F - f_hi.astype(jnp.float32)).astype(jnp.bfloat16)
    f96 = jnp.concatenate([f_hi, f_hi, f_lo], axis=0).reshape(96, _BLK)
    dn = (((0,), (0,)), ((), ()))
    both = jax.lax.dot_general(f96, t_ref[...], dn,
                               preferred_element_type=jnp.float32)
    o_ref[...] = both[:, :_NCOLS] * both[:, 128:128 + _NCOLS]


def kernel(lonlat):
    n = lonlat.shape[0]
    nb = -(-n // _BLK)
    npad = nb * _BLK
    llt = jnp.transpose(lonlat)
    if npad != n:
        llt = jnp.pad(llt, ((0, 0), (0, npad - n)))
    llt = llt.reshape(2, nb, 8, _BLK // 8)
    out = pl.pallas_call(
        _sph_body,
        grid=(nb,),
        in_specs=[
            pl.BlockSpec((2, 1, 8, _BLK // 8), lambda i: (0, i, 0, 0)),
            pl.BlockSpec((96, 256), lambda i: (0, 0)),
        ],
        out_specs=pl.BlockSpec((_BLK, _NCOLS), lambda i: (i, 0)),
        out_shape=jax.ShapeDtypeStruct((npad, _NCOLS), jnp.float32),
    )(llt, jnp.asarray(_TAB96))
    return out[:n] if npad != n else out


# speedup vs baseline: 1.0436x; 1.0436x over previous
import math

import jax
import jax.numpy as jnp
import numpy as np
from jax.experimental import pallas as pl
from jax.experimental.pallas import tpu as pltpu

_L = 10
_NCOLS = _L * _L
_KDIM = 32
_BLK = 16384


def _build_table():
    from numpy.polynomial import chebyshev as _cheb
    Q = {(0, 0): np.zeros(_L)}
    Q[(0, 0)][0] = 1.0

    def shift(q):
        r = np.zeros(_L)
        r[1:] = q[:-1]
        return r

    for m in range(1, _L):
        Q[(m, m)] = -(2.0 * m - 1.0) * Q[(m - 1, m - 1)]
    for m in range(0, _L - 1):
        Q[(m + 1, m)] = (2.0 * m + 1.0) * shift(Q[(m, m)])
    for m in range(0, _L):
        for l in range(m + 2, _L):
            Q[(l, m)] = ((2.0 * l - 1.0) * shift(Q[(l - 1, m)])
                         - (l + m - 1.0) * Q[(l - 2, m)]) / (l - m)

    tab = np.zeros((_KDIM, 256))
    j = 0
    for l in range(_L):
        for m in range(-l, l + 1):
            am = abs(m)
            K = math.sqrt((2 * l + 1) / (4.0 * math.pi)
                          * math.factorial(l - am) / math.factorial(l + am))
            fac = K * (math.sqrt(2.0) if m != 0 else 1.0)
            cheb = _cheb.poly2cheb(fac * Q[(l, am)])
            tab[0:len(cheb), j] = cheb
            tab[(10 + am) if m >= 0 else (20 + am), 128 + j] = 1.0
            j += 1
    import ml_dtypes
    tab32 = tab.astype(np.float32)
    t_hi = tab32.astype(ml_dtypes.bfloat16)
    t_lo = (tab32 - t_hi.astype(np.float32)).astype(ml_dtypes.bfloat16)
    return np.concatenate([t_hi, t_lo, t_hi], axis=0)


_TAB96 = _build_table()


def _sph_body(ll_ref, t_ref, o_ref):
    ll = ll_ref[...]
    lon = ll[0:1, :]
    lat = ll[1:2, :]
    deg = np.float32(math.pi / 180.0)
    phi = (lon + 180.0) * deg
    theta = (lat + 90.0) * deg
    x = jnp.cos(theta)
    sn = jnp.sin(theta)
    wr = sn * jnp.cos(phi)
    wi = sn * jnp.sin(phi)

    one = jnp.ones_like(x)
    zero = jnp.zeros_like(x)
    two_x = x + x
    ts = [one, x]
    for _ in range(2, _L):
        ts.append(two_x * ts[-1] - ts[-2])
    re = [one]
    im = [zero]
    for m in range(1, _L):
        re.append(re[m - 1] * wr - im[m - 1] * wi)
        im.append(re[m - 1] * wi + im[m - 1] * wr)

    F = jnp.concatenate(ts + re + im + [zero, zero], axis=0)
    f_hi = F.astype(jnp.bfloat16)
    f_lo = (F - f_hi.astype(jnp.float32)).astype(jnp.bfloat16)
    f96 = jnp.concatenate([f_hi, f_hi, f_lo], axis=0)
    dn = (((0,), (0,)), ((), ()))
    both = jax.lax.dot_general(f96, t_ref[...], dn,
                               preferred_element_type=jnp.float32)
    o_ref[...] = both[:, :_NCOLS] * both[:, 128:128 + _NCOLS]


def kernel(lonlat):
    n = lonlat.shape[0]
    nb = -(-n // _BLK)
    npad = nb * _BLK
    llt = jnp.transpose(lonlat)
    if npad != n:
        llt = jnp.pad(llt, ((0, 0), (0, npad - n)))
    out = pl.pallas_call(
        _sph_body,
        grid=(nb,),
        compiler_params=pltpu.CompilerParams(
            dimension_semantics=("parallel",)),
        in_specs=[
            pl.BlockSpec((2, _BLK), lambda i: (0, i)),
            pl.BlockSpec((96, 256), lambda i: (0, 0)),
        ],
        out_specs=pl.BlockSpec((_BLK, _NCOLS), lambda i: (i, 0)),
        out_shape=jax.ShapeDtypeStruct((npad, _NCOLS), jnp.float32),
    )(llt, jnp.asarray(_TAB96))
    return out[:n] if npad != n else out


# speedup vs baseline: 1.0855x; 1.0401x over previous
import math

import jax
import jax.numpy as jnp
import numpy as np
from jax.experimental import pallas as pl
from jax.experimental.pallas import tpu as pltpu

_L = 10
_NCOLS = _L * _L
_KDIM = 32
_BLK = 16384


def _build_table():
    from numpy.polynomial import chebyshev as _cheb
    Q = {(0, 0): np.zeros(_L)}
    Q[(0, 0)][0] = 1.0

    def shift(q):
        r = np.zeros(_L)
        r[1:] = q[:-1]
        return r

    for m in range(1, _L):
        Q[(m, m)] = -(2.0 * m - 1.0) * Q[(m - 1, m - 1)]
    for m in range(0, _L - 1):
        Q[(m + 1, m)] = (2.0 * m + 1.0) * shift(Q[(m, m)])
    for m in range(0, _L):
        for l in range(m + 2, _L):
            Q[(l, m)] = ((2.0 * l - 1.0) * shift(Q[(l - 1, m)])
                         - (l + m - 1.0) * Q[(l - 2, m)]) / (l - m)

    tab = np.zeros((_KDIM, 256))
    j = 0
    for l in range(_L):
        for m in range(-l, l + 1):
            am = abs(m)
            K = math.sqrt((2 * l + 1) / (4.0 * math.pi)
                          * math.factorial(l - am) / math.factorial(l + am))
            fac = K * (math.sqrt(2.0) if m != 0 else 1.0)
            cheb = _cheb.poly2cheb(fac * Q[(l, am)])
            tab[0:len(cheb), j] = cheb
            tab[(10 + am) if m >= 0 else (20 + am), 128 + j] = 1.0
            j += 1
    import ml_dtypes
    tab32 = tab.astype(np.float32)
    t_hi = tab32.astype(ml_dtypes.bfloat16)
    t_lo = (tab32 - t_hi.astype(np.float32)).astype(ml_dtypes.bfloat16)
    return np.concatenate([t_hi, t_lo, t_hi], axis=0)


_TAB96 = _build_table()


def _sph_body(ll_ref, t_ref, o_ref):
    ll = ll_ref[...]
    lon = ll[0:1, :]
    lat = ll[1:2, :]
    deg = np.float32(math.pi / 180.0)
    phi = (lon + 180.0) * deg
    theta = (lat + 90.0) * deg
    x = jnp.cos(theta)
    sn = jnp.sin(theta)
    wr = sn * jnp.cos(phi)
    wi = sn * jnp.sin(phi)

    one = jnp.ones_like(x)
    zero = jnp.zeros_like(x)
    two_x = x + x
    ts = [one, x]
    for _ in range(2, _L):
        ts.append(two_x * ts[-1] - ts[-2])
    re = [one]
    im = [zero]
    for m in range(1, _L):
        re.append(re[m - 1] * wr - im[m - 1] * wi)
        im.append(re[m - 1] * wi + im[m - 1] * wr)

    F = jnp.concatenate(ts + re + im + [zero, zero], axis=0)
    f_hi = F.astype(jnp.bfloat16)
    f_lo = (F - f_hi.astype(jnp.float32)).astype(jnp.bfloat16)
    f96 = jnp.concatenate([f_hi, f_hi, f_lo], axis=0)
    dn = (((0,), (0,)), ((), ()))
    both = jax.lax.dot_general(f96, t_ref[...], dn,
                               preferred_element_type=jnp.float32)
    o_ref[...] = both[:, :128] * both[:, 128:256]


def kernel(lonlat):
    n = lonlat.shape[0]
    nb = -(-n // _BLK)
    npad = nb * _BLK
    llt = jnp.transpose(lonlat)
    if npad != n:
        llt = jnp.pad(llt, ((0, 0), (0, npad - n)))
    out = pl.pallas_call(
        _sph_body,
        grid=(nb,),
        compiler_params=pltpu.CompilerParams(
            dimension_semantics=("parallel",)),
        in_specs=[
            pl.BlockSpec((2, _BLK), lambda i: (0, i)),
            pl.BlockSpec((96, 256), lambda i: (0, 0)),
        ],
        out_specs=pl.BlockSpec((_BLK, 128), lambda i: (i, 0)),
        out_shape=jax.ShapeDtypeStruct((npad, 128), jnp.float32),
    )(llt, jnp.asarray(_TAB96))
    return out[:n, :_NCOLS]
